# Initial kernel scaffold; baseline (speedup 1.0000x reference)
import jax
import jax.numpy as jnp
from jax.experimental import pallas as pl


def kernel(atom_embedding, edge_index, edge_embedding, W, b):
    raise NotImplementedError("write your pallas kernel here")


# speedup vs baseline: 1.5948x; 1.5948x over previous
import functools

import jax
import jax.numpy as jnp
from jax import lax
from jax.experimental import pallas as pl
from jax.experimental.pallas import tpu as pltpu
from jax.experimental.pallas import tpu_sc as plsc

N_NODES = 10000
HIDDEN = 128
EDGE_DIM = 64
HID_I32 = HIDDEN // 2

NC, NS = 2, 16
NW = NC * NS
CHUNK = 128

EDGE_BLOCK = 2560


def _sc_gather(table, idx):
    e_pad = idx.shape[0]
    per_w = e_pad // NW
    n_chunks = per_w // CHUNK
    mesh = plsc.VectorSubcoreMesh(core_axis_name="c", subcore_axis_name="s")

    @functools.partial(
        pl.kernel,
        out_type=jax.ShapeDtypeStruct((e_pad, HIDDEN), jnp.float32),
        mesh=mesh,
        scratch_types=[
            pltpu.VMEM((per_w,), jnp.int32),
            pltpu.VMEM((CHUNK, HIDDEN), jnp.float32),
            pltpu.SemaphoreType.DMA,
        ],
    )
    def gather_kernel(table_hbm, idx_hbm, out_hbm, idx_v, rows_v, sem):
        wid = lax.axis_index("s") * NC + lax.axis_index("c")
        base = wid * per_w
        pltpu.sync_copy(idx_hbm.at[pl.ds(base, per_w)], idx_v)

        def body(j, _):
            pltpu.async_copy(
                table_hbm.at[idx_v.at[pl.ds(j * CHUNK, CHUNK)]],
                rows_v, sem).wait()
            pltpu.sync_copy(rows_v, out_hbm.at[pl.ds(base + j * CHUNK, CHUNK)])
            return 0

        lax.fori_loop(0, n_chunks, body, 0)

    return gather_kernel(table, idx)


def _tc_linear(g_bf16, edge_embedding, w_top, w_bot, b):
    n_edges = edge_embedding.shape[0]
    grid = (n_edges // EDGE_BLOCK,)

    def body(g_ref, e_ref, wt_ref, wb_ref, b_ref, o_ref):
        g = g_ref[...].astype(jnp.float32)
        acc = jnp.dot(g, wt_ref[...], preferred_element_type=jnp.float32)
        acc = acc + jnp.dot(e_ref[...], wb_ref[...],
                            preferred_element_type=jnp.float32)
        o_ref[...] = jnp.maximum(acc + b_ref[...], 0.0)

    return pl.pallas_call(
        body,
        grid=grid,
        in_specs=[
            pl.BlockSpec((EDGE_BLOCK, HIDDEN), lambda i: (i, 0)),
            pl.BlockSpec((EDGE_BLOCK, EDGE_DIM), lambda i: (i, 0)),
            pl.BlockSpec((HIDDEN, HIDDEN), lambda i: (0, 0)),
            pl.BlockSpec((EDGE_DIM, HIDDEN), lambda i: (0, 0)),
            pl.BlockSpec((1, HIDDEN), lambda i: (0, 0)),
        ],
        out_specs=pl.BlockSpec((EDGE_BLOCK, HIDDEN), lambda i: (i, 0)),
        out_shape=jax.ShapeDtypeStruct((n_edges, HIDDEN), jnp.float32),
    )(g_bf16, edge_embedding, w_top, w_bot, b)


def kernel(atom_embedding, edge_index, edge_embedding, W, b):
    n_edges = edge_index.shape[1]
    gran = NW * CHUNK
    e_pad = ((n_edges + gran - 1) // gran) * gran

    src = edge_index[0]
    if e_pad != n_edges:
        src = jnp.concatenate(
            [src, jnp.zeros((e_pad - n_edges,), dtype=src.dtype)])

    g = _sc_gather(atom_embedding, src)

    out = _tc_linear(g, edge_embedding, W[:HIDDEN], W[HIDDEN:],
                     b.reshape(1, HIDDEN))
    return out


# speedup vs baseline: 2.4001x; 1.5049x over previous
import functools

import jax
import jax.numpy as jnp
from jax import lax
from jax.experimental import pallas as pl
from jax.experimental.pallas import tpu as pltpu
from jax.experimental.pallas import tpu_sc as plsc

N_NODES = 10000
HIDDEN = 128
EDGE_DIM = 64
HID_I32 = HIDDEN // 2

NC, NS = 2, 16
NW = NC * NS
CHUNK = 128
KDEPTH = 4

EDGE_BLOCK = 2560


def _sc_gather(table, idx):
    e_pad = idx.shape[0]
    per_w = e_pad // NW
    n_chunks = per_w // CHUNK
    mesh = plsc.VectorSubcoreMesh(core_axis_name="c", subcore_axis_name="s")

    @functools.partial(
        pl.kernel,
        out_type=jax.ShapeDtypeStruct((e_pad, HIDDEN), jnp.float32),
        mesh=mesh,
        scratch_types=[
            pltpu.VMEM((per_w,), jnp.int32),
            pltpu.VMEM((CHUNK, HIDDEN), jnp.float32),
            pltpu.SemaphoreType.DMA,
        ],
    )
    def gather_kernel(table_hbm, idx_hbm, out_hbm, idx_v, rows_v, sem):
        wid = lax.axis_index("s") * NC + lax.axis_index("c")
        base = wid * per_w
        pltpu.sync_copy(idx_hbm.at[pl.ds(base, per_w)], idx_v)

        def body(j, _):
            pltpu.async_copy(
                table_hbm.at[idx_v.at[pl.ds(j * CHUNK, CHUNK)]],
                rows_v, sem).wait()
            pltpu.sync_copy(rows_v, out_hbm.at[pl.ds(base + j * CHUNK, CHUNK)])
            return 0

        lax.fori_loop(0, n_chunks, body, 0)

    return gather_kernel(table, idx)


def _tc_linear(g_bf16, edge_embedding, w_top, w_bot, b):
    n_edges = edge_embedding.shape[0]
    grid = (n_edges // EDGE_BLOCK,)

    def body(g_ref, e_ref, wt_ref, wb_ref, b_ref, o_ref):
        g = g_ref[...].astype(jnp.float32)
        acc = jnp.dot(g, wt_ref[...], preferred_element_type=jnp.float32)
        acc = acc + jnp.dot(e_ref[...], wb_ref[...],
                            preferred_element_type=jnp.float32)
        o_ref[...] = jnp.maximum(acc + b_ref[...], 0.0)

    return pl.pallas_call(
        body,
        grid=grid,
        in_specs=[
            pl.BlockSpec((EDGE_BLOCK, HIDDEN), lambda i: (i, 0)),
            pl.BlockSpec((EDGE_BLOCK, EDGE_DIM), lambda i: (i, 0)),
            pl.BlockSpec((HIDDEN, HIDDEN), lambda i: (0, 0)),
            pl.BlockSpec((EDGE_DIM, HIDDEN), lambda i: (0, 0)),
            pl.BlockSpec((1, HIDDEN), lambda i: (0, 0)),
        ],
        out_specs=pl.BlockSpec((EDGE_BLOCK, HIDDEN), lambda i: (i, 0)),
        out_shape=jax.ShapeDtypeStruct((n_edges, HIDDEN), jnp.float32),
    )(g_bf16, edge_embedding, w_top, w_bot, b)


def kernel(atom_embedding, edge_index, edge_embedding, W, b):
    n_edges = edge_index.shape[1]
    gran = NW * CHUNK
    e_pad = ((n_edges + gran - 1) // gran) * gran

    src = edge_index[0]
    if e_pad != n_edges:
        pad = jnp.arange(e_pad - n_edges, dtype=src.dtype) % N_NODES
        src = jnp.concatenate([src, pad])

    g = _sc_gather(atom_embedding, src)

    out = _tc_linear(g, edge_embedding, W[:HIDDEN], W[HIDDEN:],
                     b.reshape(1, HIDDEN))
    return out


# speedup vs baseline: 2.8618x; 1.1923x over previous
import functools

import jax
import jax.numpy as jnp
from jax import lax
from jax.experimental import pallas as pl
from jax.experimental.pallas import tpu as pltpu
from jax.experimental.pallas import tpu_sc as plsc

N_NODES = 10000
HIDDEN = 128
EDGE_DIM = 64
HID_I32 = HIDDEN // 2

NC, NS = 2, 16
NW = NC * NS
CHUNK = 128
KDEPTH = 4

EDGE_BLOCK = 2560


def _sc_gather(table, idx):
    n_pad = table.shape[0]
    e_pad = idx.shape[0]
    per_w = e_pad // NW
    n_chunks = per_w // CHUNK
    mesh = plsc.VectorSubcoreMesh(core_axis_name="c", subcore_axis_name="s")

    @functools.partial(
        pl.kernel,
        out_type=jax.ShapeDtypeStruct((e_pad, HIDDEN), jnp.float32),
        mesh=mesh,
        scratch_types=[
            pltpu.VMEM((per_w,), jnp.int32),
            pltpu.VMEM((CHUNK, HIDDEN), jnp.float32),
            pltpu.VMEM_SHARED((n_pad, HIDDEN), jnp.float32),
            pltpu.SemaphoreType.DMA,
        ],
    )
    def gather_kernel(table_hbm, idx_hbm, out_hbm, idx_v, rows_v, spm, sem):
        sid = lax.axis_index("s")
        wid = sid * NC + lax.axis_index("c")
        base = wid * per_w
        rows_per_sub = n_pad // NS
        pltpu.sync_copy(table_hbm.at[pl.ds(sid * rows_per_sub, rows_per_sub)],
                        spm.at[pl.ds(sid * rows_per_sub, rows_per_sub)])
        pltpu.sync_copy(idx_hbm.at[pl.ds(base, per_w)], idx_v)
        plsc.subcore_barrier()

        def body(j, _):
            pltpu.async_copy(
                spm.at[idx_v.at[pl.ds(j * CHUNK, CHUNK)]],
                rows_v, sem).wait()
            pltpu.sync_copy(rows_v, out_hbm.at[pl.ds(base + j * CHUNK, CHUNK)])
            return 0

        lax.fori_loop(0, n_chunks, body, 0)

    return gather_kernel(table, idx)


def _tc_linear(g_bf16, edge_embedding, w_top, w_bot, b):
    n_edges = edge_embedding.shape[0]
    grid = (n_edges // EDGE_BLOCK,)

    def body(g_ref, e_ref, wt_ref, wb_ref, b_ref, o_ref):
        g = g_ref[...].astype(jnp.float32)
        acc = jnp.dot(g, wt_ref[...], preferred_element_type=jnp.float32)
        acc = acc + jnp.dot(e_ref[...], wb_ref[...],
                            preferred_element_type=jnp.float32)
        o_ref[...] = jnp.maximum(acc + b_ref[...], 0.0)

    return pl.pallas_call(
        body,
        grid=grid,
        in_specs=[
            pl.BlockSpec((EDGE_BLOCK, HIDDEN), lambda i: (i, 0)),
            pl.BlockSpec((EDGE_BLOCK, EDGE_DIM), lambda i: (i, 0)),
            pl.BlockSpec((HIDDEN, HIDDEN), lambda i: (0, 0)),
            pl.BlockSpec((EDGE_DIM, HIDDEN), lambda i: (0, 0)),
            pl.BlockSpec((1, HIDDEN), lambda i: (0, 0)),
        ],
        out_specs=pl.BlockSpec((EDGE_BLOCK, HIDDEN), lambda i: (i, 0)),
        out_shape=jax.ShapeDtypeStruct((n_edges, HIDDEN), jnp.float32),
    )(g_bf16, edge_embedding, w_top, w_bot, b)


def kernel(atom_embedding, edge_index, edge_embedding, W, b):
    n_edges = edge_index.shape[1]
    gran = NW * CHUNK
    e_pad = ((n_edges + gran - 1) // gran) * gran

    src = edge_index[0]
    if e_pad != n_edges:
        pad = jnp.arange(e_pad - n_edges, dtype=src.dtype) % N_NODES
        src = jnp.concatenate([src, pad])

    n_pad = ((N_NODES + 8 * NS - 1) // (8 * NS)) * (8 * NS)
    table = jnp.concatenate(
        [atom_embedding,
         jnp.zeros((n_pad - N_NODES, HIDDEN), atom_embedding.dtype)])

    g = _sc_gather(table, src)

    out = _tc_linear(g, edge_embedding, W[:HIDDEN], W[HIDDEN:],
                     b.reshape(1, HIDDEN))
    return out


# speedup vs baseline: 3.1333x; 1.0949x over previous
import functools

import jax
import jax.numpy as jnp
from jax import lax
from jax.experimental import pallas as pl
from jax.experimental.pallas import tpu as pltpu
from jax.experimental.pallas import tpu_sc as plsc

N_NODES = 10000
HIDDEN = 128
EDGE_DIM = 64
HID_I32 = HIDDEN // 2

NC, NS = 2, 16
NW = NC * NS
CHUNK = 80

EDGE_BLOCK = 6400


def _sc_gather(table, idx):
    n_pad = table.shape[0]
    e_pad = idx.shape[0]
    per_w = e_pad // NW
    n_chunks = per_w // CHUNK
    mesh = plsc.VectorSubcoreMesh(core_axis_name="c", subcore_axis_name="s")

    @functools.partial(
        pl.kernel,
        out_type=jax.ShapeDtypeStruct((e_pad, HIDDEN), jnp.float32),
        mesh=mesh,
        scratch_types=[
            pltpu.VMEM((per_w,), jnp.int32),
            pltpu.VMEM((CHUNK, HIDDEN), jnp.float32),
            pltpu.VMEM_SHARED((n_pad, HIDDEN), jnp.float32),
            pltpu.SemaphoreType.DMA,
        ],
    )
    def gather_kernel(table_hbm, idx_hbm, out_hbm, idx_v, rows_v, spm, sem):
        sid = lax.axis_index("s")
        wid = sid * NC + lax.axis_index("c")
        base = wid * per_w
        rows_per_sub = n_pad // NS
        pltpu.sync_copy(table_hbm.at[pl.ds(sid * rows_per_sub, rows_per_sub)],
                        spm.at[pl.ds(sid * rows_per_sub, rows_per_sub)])
        pltpu.sync_copy(idx_hbm.at[pl.ds(base, per_w)], idx_v)
        plsc.subcore_barrier()

        def body(j, _):
            pltpu.async_copy(
                spm.at[idx_v.at[pl.ds(j * CHUNK, CHUNK)]],
                rows_v, sem).wait()
            pltpu.sync_copy(rows_v, out_hbm.at[pl.ds(base + j * CHUNK, CHUNK)])
            return 0

        lax.fori_loop(0, n_chunks, body, 0)

    return gather_kernel(table, idx)


def _tc_linear(g_bf16, edge_embedding, w_top, w_bot, b):
    n_edges = edge_embedding.shape[0]
    grid = (n_edges // EDGE_BLOCK,)

    def body(g_ref, e_ref, wt_ref, wb_ref, b_ref, o_ref):
        g = g_ref[...].astype(jnp.float32)
        acc = jnp.dot(g, wt_ref[...], preferred_element_type=jnp.float32)
        acc = acc + jnp.dot(e_ref[...], wb_ref[...],
                            preferred_element_type=jnp.float32)
        o_ref[...] = jnp.maximum(acc + b_ref[...], 0.0)

    return pl.pallas_call(
        body,
        grid=grid,
        in_specs=[
            pl.BlockSpec((EDGE_BLOCK, HIDDEN), lambda i: (i, 0)),
            pl.BlockSpec((EDGE_BLOCK, EDGE_DIM), lambda i: (i, 0)),
            pl.BlockSpec((HIDDEN, HIDDEN), lambda i: (0, 0)),
            pl.BlockSpec((EDGE_DIM, HIDDEN), lambda i: (0, 0)),
            pl.BlockSpec((1, HIDDEN), lambda i: (0, 0)),
        ],
        out_specs=pl.BlockSpec((EDGE_BLOCK, HIDDEN), lambda i: (i, 0)),
        out_shape=jax.ShapeDtypeStruct((n_edges, HIDDEN), jnp.float32),
    )(g_bf16, edge_embedding, w_top, w_bot, b)


def kernel(atom_embedding, edge_index, edge_embedding, W, b):
    n_edges = edge_index.shape[1]
    gran = NW * CHUNK
    e_pad = ((n_edges + gran - 1) // gran) * gran

    src = edge_index[0]
    if e_pad != n_edges:
        pad = jnp.arange(e_pad - n_edges, dtype=src.dtype) % N_NODES
        src = jnp.concatenate([src, pad])

    n_pad = ((N_NODES + 8 * NS - 1) // (8 * NS)) * (8 * NS)
    table = jnp.concatenate(
        [atom_embedding,
         jnp.zeros((n_pad - N_NODES, HIDDEN), atom_embedding.dtype)])

    g = _sc_gather(table, src)

    out = _tc_linear(g, edge_embedding, W[:HIDDEN], W[HIDDEN:],
                     b.reshape(1, HIDDEN))
    return out
